# baseline (device time: 37794 ns/iter reference)
import jax
import jax.numpy as jnp
from jax import lax
from jax.experimental import pallas as pl
from jax.experimental.pallas import tpu as pltpu

T = 256
D = 512
V_SHARD = 4096
V_GLOBAL = 8192


def kernel(x, W):
    def body(x_ref, w_ref, out_ref, send_buf, recv_buf, send_sem, recv_sem):
        my_x = lax.axis_index("x")
        my_y = lax.axis_index("y")
        peer = (1 - my_x, my_y)

        barrier = pltpu.get_barrier_semaphore()
        pl.semaphore_signal(
            barrier, inc=1, device_id=peer, device_id_type=pl.DeviceIdType.MESH
        )
        pl.semaphore_wait(barrier, 1)

        logits = jnp.dot(
            x_ref[...].astype(jnp.bfloat16),
            w_ref[...].astype(jnp.bfloat16),
            preferred_element_type=jnp.float32,
        )
        send_buf[...] = logits.astype(jnp.bfloat16)

        rdma = pltpu.make_async_remote_copy(
            src_ref=send_buf,
            dst_ref=recv_buf,
            send_sem=send_sem,
            recv_sem=recv_sem,
            device_id=peer,
            device_id_type=pl.DeviceIdType.MESH,
        )
        rdma.start()
        rdma.wait()

        remote = recv_buf[...].astype(jnp.float32)
        m = jnp.maximum(
            jnp.max(logits, axis=-1, keepdims=True),
            jnp.max(remote, axis=-1, keepdims=True),
        )
        e_loc = jnp.exp(logits - m)
        e_rem = jnp.exp(remote - m)
        s = jnp.sum(e_loc, -1, keepdims=True) + jnp.sum(e_rem, -1, keepdims=True)
        out_ref[:, pl.ds(my_x * V_SHARD, V_SHARD)] = e_loc / s
        out_ref[:, pl.ds((1 - my_x) * V_SHARD, V_SHARD)] = e_rem / s

    return pl.pallas_call(
        body,
        out_shape=jax.ShapeDtypeStruct((T, V_GLOBAL), jnp.float32),
        in_specs=[
            pl.BlockSpec(memory_space=pltpu.VMEM),
            pl.BlockSpec(memory_space=pltpu.VMEM),
        ],
        out_specs=pl.BlockSpec(memory_space=pltpu.VMEM),
        scratch_shapes=[
            pltpu.VMEM((T, V_SHARD), jnp.bfloat16),
            pltpu.VMEM((T, V_SHARD), jnp.bfloat16),
            pltpu.SemaphoreType.DMA,
            pltpu.SemaphoreType.DMA,
        ],
        compiler_params=pltpu.CompilerParams(collective_id=0),
    )(x, W)


# device time: 35316 ns/iter; 1.0702x vs baseline; 1.0702x over previous
import jax
import jax.numpy as jnp
from jax import lax
from jax.experimental import pallas as pl
from jax.experimental.pallas import tpu as pltpu

T = 256
D = 512
V_SHARD = 4096
V_GLOBAL = 8192
K = 8
R = T // K


def kernel(x, W):
    def body(x_ref, w_ref, out_ref, send_buf, recv_buf, send_sems, recv_sems):
        my_x = lax.axis_index("x")
        my_y = lax.axis_index("y")
        peer = (1 - my_x, my_y)

        barrier = pltpu.get_barrier_semaphore()
        pl.semaphore_signal(
            barrier, inc=1, device_id=peer, device_id_type=pl.DeviceIdType.MESH
        )
        pl.semaphore_wait(barrier, 1)

        def chunk_rdma(k):
            return pltpu.make_async_remote_copy(
                src_ref=send_buf.at[k],
                dst_ref=recv_buf.at[k],
                send_sem=send_sems.at[k],
                recv_sem=recv_sems.at[k],
                device_id=peer,
                device_id_type=pl.DeviceIdType.MESH,
            )

        xb = x_ref[...].astype(jnp.bfloat16)

        for k in range(K):
            lg = jnp.dot(
                xb[k * R : (k + 1) * R, :],
                w_ref[...].astype(jnp.bfloat16),
                preferred_element_type=jnp.float32,
            )
            send_buf[k] = lg.astype(jnp.bfloat16)
            chunk_rdma(k).start()

        for k in range(K):
            rdma = chunk_rdma(k)
            rdma.wait_recv()
            loc = send_buf[k].astype(jnp.float32)
            rem = recv_buf[k].astype(jnp.float32)
            m = jnp.maximum(
                jnp.max(loc, axis=-1, keepdims=True),
                jnp.max(rem, axis=-1, keepdims=True),
            )
            e_loc = jnp.exp(loc - m)
            e_rem = jnp.exp(rem - m)
            s = jnp.sum(e_loc, -1, keepdims=True) + jnp.sum(e_rem, -1, keepdims=True)
            rows = pl.ds(k * R, R)
            out_ref[rows, pl.ds(my_x * V_SHARD, V_SHARD)] = e_loc / s
            out_ref[rows, pl.ds((1 - my_x) * V_SHARD, V_SHARD)] = e_rem / s
            rdma.wait_send()

    return pl.pallas_call(
        body,
        out_shape=jax.ShapeDtypeStruct((T, V_GLOBAL), jnp.float32),
        in_specs=[
            pl.BlockSpec(memory_space=pltpu.VMEM),
            pl.BlockSpec(memory_space=pltpu.VMEM),
        ],
        out_specs=pl.BlockSpec(memory_space=pltpu.VMEM),
        scratch_shapes=[
            pltpu.VMEM((K, R, V_SHARD), jnp.bfloat16),
            pltpu.VMEM((K, R, V_SHARD), jnp.bfloat16),
            pltpu.SemaphoreType.DMA((K,)),
            pltpu.SemaphoreType.DMA((K,)),
        ],
        compiler_params=pltpu.CompilerParams(collective_id=0),
    )(x, W)
